# baseline (speedup 1.0000x reference)
import functools

import jax
import jax.numpy as jnp
from jax import lax
from jax.experimental import pallas as pl
from jax.experimental.pallas import tpu as pltpu
from jax.experimental.pallas import tpu_sc as plsc

TOPK = 3
COMMIT = 0.25
BIG_ID = float(2**30)
CHUNK = 128


def _top3_merge(c, cid):
    mins, sels = [], []
    for _ in range(TOPK):
        m = jnp.min(c, axis=1, keepdims=True)
        sel = jnp.min(jnp.where(c == m, cid, BIG_ID), axis=1, keepdims=True)
        mins.append(m)
        sels.append(sel)
        c = jnp.where(cid == sel, jnp.inf, c)
    return mins, sels


def _pack3(mins, sels, R):
    lane = lax.broadcasted_iota(jnp.int32, (R, 128), 1)
    nv = jnp.full((R, 128), jnp.inf, jnp.float32)
    ni = jnp.full((R, 128), BIG_ID, jnp.float32)
    for t in range(TOPK - 1, -1, -1):
        nv = jnp.where(lane == t, mins[t], nv)
        ni = jnp.where(lane == t, sels[t], ni)
    return nv, ni


def _dist_body(nk, x_ref, w_ref, d_ref, cid_ref, vals_scr, ids_scr):
    k = pl.program_id(1)
    R = x_ref.shape[0]
    KB = w_ref.shape[0]
    nch = KB // CHUNK

    x = x_ref[...]
    w = w_ref[...]
    xx = jnp.sum(x * x, axis=1, keepdims=True)
    ww = jnp.sum(w * w, axis=1)[None, :]
    xw = lax.dot_general(x, w, (((1,), (1,)), ((), ())),
                         preferred_element_type=jnp.float32)
    d = xx + ww - 2.0 * xw
    d_ref[...] = d

    @pl.when(k == 0)
    def _init():
        vals_scr[...] = jnp.full((R, 128), jnp.inf, jnp.float32)
        ids_scr[...] = jnp.full((R, 128), BIG_ID, jnp.float32)

    cms = [jnp.min(d[:, j * CHUNK:(j + 1) * CHUNK], axis=1, keepdims=True)
           for j in range(nch)]
    cm = jnp.concatenate(cms, axis=1)
    cmid = (lax.broadcasted_iota(jnp.int32, (R, nch), 1)
            + k * nch).astype(jnp.float32)

    c = jnp.concatenate([cm, vals_scr[...]], axis=1)
    cid = jnp.concatenate([cmid, ids_scr[...]], axis=1)
    mins, sels = _top3_merge(c, cid)
    nv, ni = _pack3(mins, sels, R)
    vals_scr[...] = nv
    ids_scr[...] = ni

    @pl.when(k == nk - 1)
    def _emit():
        cid_ref[...] = ids_scr[...].astype(jnp.int32)


def _dist_chunktop3(flat, emb, R=2048, KB=1024):
    N, C = flat.shape
    K = emb.shape[0]
    nr, nk = N // R, K // KB
    body = functools.partial(_dist_body, nk)
    return pl.pallas_call(
        body,
        grid=(nr, nk),
        in_specs=[
            pl.BlockSpec((R, C), lambda r, k: (r, 0)),
            pl.BlockSpec((KB, C), lambda r, k: (k, 0)),
        ],
        out_specs=[
            pl.BlockSpec((R, KB), lambda r, k: (r, k)),
            pl.BlockSpec((R, 128), lambda r, k: (r, 0)),
        ],
        out_shape=[
            jax.ShapeDtypeStruct((N, K), jnp.float32),
            jax.ShapeDtypeStruct((N, 128), jnp.int32),
        ],
        scratch_shapes=[
            pltpu.VMEM((R, 128), jnp.float32),
            pltpu.VMEM((R, 128), jnp.float32),
        ],
    )(flat, emb)


def _sc_gather(table, idx_flat, hist_K=None):
    info = plsc.get_sparse_core_info()
    NC, NS, L = info.num_cores, info.num_subcores, info.num_lanes
    NW = NC * NS
    M = idx_flat.shape[0]
    C = table.shape[1]
    per_w = M // NW
    CH = 128
    n_ch = per_w // CH

    out_type = [jax.ShapeDtypeStruct((M, C), jnp.float32)]
    scratch = [
        pltpu.VMEM((CH,), jnp.int32),
        pltpu.VMEM((CH, C), jnp.float32),
        pltpu.SemaphoreType.DMA,
    ]
    if hist_K is not None:
        out_type.append(jax.ShapeDtypeStruct((NW, hist_K), jnp.float32))
        scratch.append(pltpu.VMEM((hist_K,), jnp.float32))

    mesh = plsc.VectorSubcoreMesh(core_axis_name="c", subcore_axis_name="s")

    @functools.partial(
        pl.kernel,
        out_type=out_type,
        mesh=mesh,
        scratch_types=scratch,
        compiler_params=pltpu.CompilerParams(needs_layout_passes=False),
    )
    def k(table_hbm, idx_hbm, *refs):
        if hist_K is not None:
            outq_hbm, cnt_hbm, idx_v, rows_v, sem, cnt_v = refs
        else:
            outq_hbm, idx_v, rows_v, sem = refs
        cid = lax.axis_index("c")
        sid = lax.axis_index("s")
        wid = sid * NC + cid
        base = wid * per_w

        if hist_K is not None:
            def zero_body(i, _):
                cnt_v[pl.ds(i * L, L)] = jnp.zeros((L,), jnp.float32)
                return 0
            lax.fori_loop(0, hist_K // L, zero_body, 0)

        ones = jnp.ones((L,), jnp.float32)
        for c in range(n_ch):
            pltpu.sync_copy(idx_hbm.at[pl.ds(base + c * CH, CH)], idx_v)
            pltpu.async_copy(table_hbm.at[idx_v], rows_v, sem).wait()
            pltpu.sync_copy(rows_v, outq_hbm.at[pl.ds(base + c * CH, CH)])
            if hist_K is not None:
                for j in range(CH // L):
                    v = idx_v[pl.ds(j * L, L)]
                    plsc.addupdate_scatter(cnt_v, [v], ones)

        if hist_K is not None:
            pltpu.sync_copy(cnt_v, cnt_hbm.at[wid])

    return k(table, idx_flat)


def _exact_body(C_feat, cand_ref, ch_ref, idx_ref, loss_ref, loss_scr):
    r = pl.program_id(0)
    R = cand_ref.shape[0]

    @pl.when(r == 0)
    def _init():
        loss_scr[0] = 0.0

    ch = ch_ref[...].astype(jnp.float32)
    iota = lax.broadcasted_iota(jnp.int32, (R, CHUNK), 1).astype(jnp.float32)
    gids = jnp.concatenate(
        [ch[:, t:t + 1] * CHUNK + iota for t in range(TOPK)], axis=1)
    c = cand_ref[...]
    mins, sels = _top3_merge(c, gids)
    nv, ni = _pack3(mins, sels, R)
    lane = lax.broadcasted_iota(jnp.int32, (R, 128), 1)
    idx_ref[...] = jnp.where(lane < TOPK, ni, 0.0).astype(jnp.int32)
    loss_scr[0] += jnp.sum(jnp.where(lane < TOPK, nv, 0.0))

    @pl.when(r == pl.num_programs(0) - 1)
    def _emit():
        n_total = R * pl.num_programs(0)
        v = COMMIT * loss_scr[0] / (n_total * TOPK * C_feat)
        loss_ref[...] = jnp.reshape(v, (1, 1))


def _exact_top3(cand, chunkids, C_feat, R=1024):
    N = cand.shape[0]
    W = cand.shape[1]
    nr = N // R
    return pl.pallas_call(
        functools.partial(_exact_body, C_feat),
        grid=(nr,),
        in_specs=[
            pl.BlockSpec((R, W), lambda r: (r, 0)),
            pl.BlockSpec((R, 128), lambda r: (r, 0)),
        ],
        out_specs=[
            pl.BlockSpec((R, 128), lambda r: (r, 0)),
            pl.BlockSpec((1, 1), lambda r: (0, 0)),
        ],
        out_shape=[
            jax.ShapeDtypeStruct((N, 128), jnp.int32),
            jax.ShapeDtypeStruct((1, 1), jnp.float32),
        ],
        scratch_shapes=[pltpu.SMEM((1,), jnp.float32)],
    )(cand, chunkids)


def _finalize_body(n_total, cnt_ref, avg_ref, perp_ref):
    counts = jnp.sum(cnt_ref[...], axis=0, keepdims=True)
    avg = counts / n_total
    avg_ref[...] = avg
    ent = jnp.sum(avg * jnp.log(avg + 1e-10))
    perp_ref[...] = jnp.reshape(jnp.exp(-ent), (1, 1))


def _finalize(cnt, n_total):
    NW, K = cnt.shape
    return pl.pallas_call(
        functools.partial(_finalize_body, n_total),
        out_specs=[
            pl.BlockSpec((1, K), lambda: (0, 0)),
            pl.BlockSpec((1, 1), lambda: (0, 0)),
        ],
        out_shape=[
            jax.ShapeDtypeStruct((1, K), jnp.float32),
            jax.ShapeDtypeStruct((1, 1), jnp.float32),
        ],
    )(cnt)


def kernel(inputs, embedding_weight):
    B, T, C = inputs.shape
    K = embedding_weight.shape[0]
    N = B * T
    flat = inputs.reshape(N, C)

    distances, chid_pad = _dist_chunktop3(flat, embedding_weight)
    chid3 = chid_pad[:, :TOPK]
    n_chunks = K // CHUNK
    grow = (jnp.arange(N, dtype=jnp.int32)[:, None] * n_chunks
            + chid3).reshape(-1)

    dist_rows = distances.reshape(N * n_chunks, CHUNK)
    (cand_flat,) = _sc_gather(dist_rows, grow)
    cand = cand_flat.reshape(N, TOPK * CHUNK)

    idx_pad, loss = _exact_top3(cand, chid_pad, C)
    idx3 = idx_pad[:, :TOPK]
    idx_flat = idx3.reshape(-1)

    quantized_flat, cnt = _sc_gather(embedding_weight, idx_flat, hist_K=K)
    avg, perp = _finalize(cnt, N)

    loss_out = loss.reshape(())
    quantized_st = quantized_flat.reshape(B, T, TOPK, C)
    perplexity = perp.reshape(())
    avg_probs = avg.reshape(K)
    encoding_indices_out = idx3.reshape(B, T, TOPK)
    distances_out = distances.reshape(B, T, K)
    return (loss_out, quantized_st, perplexity, avg_probs,
            encoding_indices_out, distances_out)


# speedup vs baseline: 1.5374x; 1.5374x over previous
import functools

import jax
import jax.numpy as jnp
from jax import lax
from jax.experimental import pallas as pl
from jax.experimental.pallas import tpu as pltpu
from jax.experimental.pallas import tpu_sc as plsc

TOPK = 3
COMMIT = 0.25
BIG_ID = float(2**30)
CHUNK = 128


def _top3_merge(c, cid):
    mins, sels = [], []
    for _ in range(TOPK):
        m = jnp.min(c, axis=1, keepdims=True)
        sel = jnp.min(jnp.where(c == m, cid, BIG_ID), axis=1, keepdims=True)
        mins.append(m)
        sels.append(sel)
        c = jnp.where(cid == sel, jnp.inf, c)
    return mins, sels


def _pack3(mins, sels, R, base=0):
    lane = lax.broadcasted_iota(jnp.int32, (R, 128), 1)
    nv = jnp.full((R, 128), jnp.inf, jnp.float32)
    ni = jnp.full((R, 128), BIG_ID, jnp.float32)
    for t in range(TOPK - 1, -1, -1):
        nv = jnp.where(lane == base + t, mins[t], nv)
        ni = jnp.where(lane == base + t, sels[t], ni)
    return nv, ni


def _dist_body(nk, x_ref, w_ref, d_ref, idx_ref, loss_ref,
               vals_scr, ids_scr, loss_scr):
    r = pl.program_id(0)
    k = pl.program_id(1)
    R = x_ref.shape[0]
    KB = w_ref.shape[0]
    nch = KB // CHUNK

    x = x_ref[...]
    w = w_ref[...]
    xx = jnp.sum(x * x, axis=1, keepdims=True)
    ww = jnp.sum(w * w, axis=1)[None, :]
    xw = lax.dot_general(x, w, (((1,), (1,)), ((), ())),
                         preferred_element_type=jnp.float32)
    d = xx + ww - 2.0 * xw
    d_ref[...] = d

    @pl.when(k == 0)
    def _init():
        vals_scr[...] = jnp.full((R, 128), jnp.inf, jnp.float32)
        ids_scr[...] = jnp.full((R, 128), BIG_ID, jnp.float32)

    @pl.when(jnp.logical_and(r == 0, k == 0))
    def _init_loss():
        loss_scr[0] = 0.0

    chunks = [d[:, j * CHUNK:(j + 1) * CHUNK] for j in range(nch)]
    cm = jnp.concatenate(
        [jnp.min(cj, axis=1, keepdims=True) for cj in chunks], axis=1)
    cmid = lax.broadcasted_iota(jnp.int32, (R, nch), 1).astype(jnp.float32)
    iota128 = lax.broadcasted_iota(jnp.int32, (R, CHUNK), 1).astype(jnp.float32)

    kbase = (k * KB).astype(jnp.float32)
    prev = []
    step_vals, step_gids = [], []
    for _ in range(TOPK):
        m = jnp.min(cm, axis=1, keepdims=True)
        selc = jnp.min(jnp.where(cm == m, cmid, BIG_ID), axis=1, keepdims=True)
        sc = chunks[0]
        for j in range(1, nch):
            sc = jnp.where(selc == float(j), chunks[j], sc)
        for pselc, plane in prev:
            sc = jnp.where(
                jnp.logical_and(pselc == selc, plane == iota128), jnp.inf, sc)
        lane = jnp.min(jnp.where(sc == m, iota128, BIG_ID), axis=1,
                       keepdims=True)
        gid = kbase + selc * CHUNK + lane
        step_vals.append(m)
        step_gids.append(gid)
        prev.append((selc, lane))
        m2 = jnp.min(jnp.where(lane == iota128, jnp.inf, sc), axis=1,
                     keepdims=True)
        cm = jnp.where(cmid == selc, m2, cm)

    sv, si = _pack3(step_vals, step_gids, R, base=3)
    lane128 = lax.broadcasted_iota(jnp.int32, (R, 128), 1)
    cv = jnp.where(lane128 < TOPK, vals_scr[...], sv)
    ci = jnp.where(lane128 < TOPK, ids_scr[...], si)
    mins, sels = _top3_merge(cv, ci)
    nv, ni = _pack3(mins, sels, R)
    vals_scr[...] = nv
    ids_scr[...] = ni

    @pl.when(k == nk - 1)
    def _emit():
        idx_ref[...] = ids_scr[...].astype(jnp.int32)
        top = jnp.where(lane128 < TOPK, vals_scr[...], 0.0)
        loss_scr[0] += jnp.sum(top)

    @pl.when(jnp.logical_and(r == pl.num_programs(0) - 1, k == nk - 1))
    def _emit_loss():
        n_total = R * pl.num_programs(0)
        v = COMMIT * loss_scr[0] / (n_total * TOPK * x_ref.shape[1])
        loss_ref[...] = jnp.reshape(v, (1, 1))


def _dist_top3(flat, emb, R=2048, KB=1024):
    N, C = flat.shape
    K = emb.shape[0]
    nr, nk = N // R, K // KB
    body = functools.partial(_dist_body, nk)
    return pl.pallas_call(
        body,
        grid=(nr, nk),
        in_specs=[
            pl.BlockSpec((R, C), lambda r, k: (r, 0)),
            pl.BlockSpec((KB, C), lambda r, k: (k, 0)),
        ],
        out_specs=[
            pl.BlockSpec((R, KB), lambda r, k: (r, k)),
            pl.BlockSpec((R, 128), lambda r, k: (r, 0)),
            pl.BlockSpec((1, 1), lambda r, k: (0, 0)),
        ],
        out_shape=[
            jax.ShapeDtypeStruct((N, K), jnp.float32),
            jax.ShapeDtypeStruct((N, 128), jnp.int32),
            jax.ShapeDtypeStruct((1, 1), jnp.float32),
        ],
        scratch_shapes=[
            pltpu.VMEM((R, 128), jnp.float32),
            pltpu.VMEM((R, 128), jnp.float32),
            pltpu.SMEM((1,), jnp.float32),
        ],
    )(flat, emb)


def _gather_hist(table, idx_flat, hist_K):
    info = plsc.get_sparse_core_info()
    NC, NS, L = info.num_cores, info.num_subcores, info.num_lanes
    NW = NC * NS
    M = idx_flat.shape[0]
    C = table.shape[1]
    per_w = M // NW
    CH = 128
    n_ch = per_w // CH
    mesh = plsc.VectorSubcoreMesh(core_axis_name="c", subcore_axis_name="s")

    @functools.partial(
        pl.kernel,
        out_type=[
            jax.ShapeDtypeStruct((M, C), jnp.float32),
            jax.ShapeDtypeStruct((NW, hist_K), jnp.float32),
        ],
        mesh=mesh,
        scratch_types=[
            pltpu.VMEM((CH,), jnp.int32),
            pltpu.VMEM((CH, C), jnp.float32),
            pltpu.VMEM((hist_K,), jnp.float32),
            pltpu.SemaphoreType.DMA,
        ],
        compiler_params=pltpu.CompilerParams(needs_layout_passes=False),
    )
    def k(table_hbm, idx_hbm, outq_hbm, cnt_hbm, idx_v, rows_v, cnt_v, sem):
        cid = lax.axis_index("c")
        sid = lax.axis_index("s")
        wid = sid * NC + cid
        base = wid * per_w

        def zero_body(i, _):
            cnt_v[pl.ds(i * L, L)] = jnp.zeros((L,), jnp.float32)
            return 0
        lax.fori_loop(0, hist_K // L, zero_body, 0)

        ones = jnp.ones((L,), jnp.float32)
        for c in range(n_ch):
            pltpu.sync_copy(idx_hbm.at[pl.ds(base + c * CH, CH)], idx_v)
            pltpu.async_copy(table_hbm.at[idx_v], rows_v, sem).wait()
            pltpu.sync_copy(rows_v, outq_hbm.at[pl.ds(base + c * CH, CH)])
            for j in range(CH // L):
                v = idx_v[pl.ds(j * L, L)]
                plsc.addupdate_scatter(cnt_v, [v], ones)

        pltpu.sync_copy(cnt_v, cnt_hbm.at[wid])

    return k(table, idx_flat)


def _finalize_body(n_total, cnt_ref, avg_ref, perp_ref):
    counts = jnp.sum(cnt_ref[...], axis=0, keepdims=True)
    avg = counts / n_total
    avg_ref[...] = avg
    ent = jnp.sum(avg * jnp.log(avg + 1e-10))
    perp_ref[...] = jnp.reshape(jnp.exp(-ent), (1, 1))


def _finalize(cnt, n_total):
    NW, K = cnt.shape
    return pl.pallas_call(
        functools.partial(_finalize_body, n_total),
        out_specs=[
            pl.BlockSpec((1, K), lambda: (0, 0)),
            pl.BlockSpec((1, 1), lambda: (0, 0)),
        ],
        out_shape=[
            jax.ShapeDtypeStruct((1, K), jnp.float32),
            jax.ShapeDtypeStruct((1, 1), jnp.float32),
        ],
    )(cnt)


def kernel(inputs, embedding_weight):
    B, T, C = inputs.shape
    K = embedding_weight.shape[0]
    N = B * T
    flat = inputs.reshape(N, C)

    distances, idx_pad, loss = _dist_top3(flat, embedding_weight)
    idx3 = idx_pad[:, :TOPK]
    idx_flat = idx3.reshape(-1)

    quantized_flat, cnt = _gather_hist(embedding_weight, idx_flat, K)
    avg, perp = _finalize(cnt, N)

    loss_out = loss.reshape(())
    quantized_st = quantized_flat.reshape(B, T, TOPK, C)
    perplexity = perp.reshape(())
    avg_probs = avg.reshape(K)
    encoding_indices_out = idx3.reshape(B, T, TOPK)
    distances_out = distances.reshape(B, T, K)
    return (loss_out, quantized_st, perplexity, avg_probs,
            encoding_indices_out, distances_out)


# speedup vs baseline: 1.7822x; 1.1593x over previous
import functools

import jax
import jax.numpy as jnp
from jax import lax
from jax.experimental import pallas as pl
from jax.experimental.pallas import tpu as pltpu
from jax.experimental.pallas import tpu_sc as plsc

TOPK = 3
COMMIT = 0.25
BIG_ID = float(2**30)
CHUNK = 128


def _top3_merge(c, cid):
    mins, sels = [], []
    for _ in range(TOPK):
        m = jnp.min(c, axis=1, keepdims=True)
        sel = jnp.min(jnp.where(c == m, cid, BIG_ID), axis=1, keepdims=True)
        mins.append(m)
        sels.append(sel)
        c = jnp.where(cid == sel, jnp.inf, c)
    return mins, sels


def _pack3(mins, sels, R, base=0):
    lane = lax.broadcasted_iota(jnp.int32, (R, 128), 1)
    nv = jnp.full((R, 128), jnp.inf, jnp.float32)
    ni = jnp.full((R, 128), BIG_ID, jnp.float32)
    for t in range(TOPK - 1, -1, -1):
        nv = jnp.where(lane == base + t, mins[t], nv)
        ni = jnp.where(lane == base + t, sels[t], ni)
    return nv, ni


def _dist_body(nk, x_ref, w_ref, d_ref, idx_ref, loss_ref,
               vals_scr, ids_scr, loss_scr):
    r = pl.program_id(0)
    k = pl.program_id(1)
    R = x_ref.shape[0]
    KB = w_ref.shape[0]
    nch = KB // CHUNK

    x = x_ref[...]
    w = w_ref[...]
    xx = jnp.sum(x * x, axis=1, keepdims=True)
    ww = jnp.sum(w * w, axis=1)[None, :]
    xw2 = lax.dot_general(-2.0 * x, w, (((1,), (1,)), ((), ())),
                          preferred_element_type=jnp.float32)
    d = (xx + ww) + xw2
    d_ref[...] = d

    @pl.when(k == 0)
    def _init():
        vals_scr[...] = jnp.full((R, 128), jnp.inf, jnp.float32)
        ids_scr[...] = jnp.full((R, 128), BIG_ID, jnp.float32)

    @pl.when(jnp.logical_and(r == 0, k == 0))
    def _init_loss():
        loss_scr[0] = 0.0

    gids = (lax.broadcasted_iota(jnp.int32, (R, KB), 1)
            + k * KB).astype(jnp.float32)
    step_vals, step_gids = _top3_merge(d, gids)

    sv, si = _pack3(step_vals, step_gids, R, base=3)
    lane128 = lax.broadcasted_iota(jnp.int32, (R, 128), 1)
    cv = jnp.where(lane128 < TOPK, vals_scr[...], sv)
    ci = jnp.where(lane128 < TOPK, ids_scr[...], si)
    mins, sels = _top3_merge(cv, ci)
    nv, ni = _pack3(mins, sels, R)
    vals_scr[...] = nv
    ids_scr[...] = ni

    @pl.when(k == nk - 1)
    def _emit():
        idx_ref[...] = ids_scr[...].astype(jnp.int32)
        top = jnp.where(lane128 < TOPK, vals_scr[...], 0.0)
        loss_scr[0] += jnp.sum(top)

    @pl.when(jnp.logical_and(r == pl.num_programs(0) - 1, k == nk - 1))
    def _emit_loss():
        n_total = R * pl.num_programs(0)
        v = COMMIT * loss_scr[0] / (n_total * TOPK * x_ref.shape[1])
        loss_ref[...] = jnp.reshape(v, (1, 1))


def _dist_top3(flat, emb, R=2048, KB=1024):
    N, C = flat.shape
    K = emb.shape[0]
    nr, nk = N // R, K // KB
    body = functools.partial(_dist_body, nk)
    return pl.pallas_call(
        body,
        grid=(nr, nk),
        in_specs=[
            pl.BlockSpec((R, C), lambda r, k: (r, 0)),
            pl.BlockSpec((KB, C), lambda r, k: (k, 0)),
        ],
        out_specs=[
            pl.BlockSpec((R, KB), lambda r, k: (r, k)),
            pl.BlockSpec((R, 128), lambda r, k: (r, 0)),
            pl.BlockSpec((1, 1), lambda r, k: (0, 0)),
        ],
        out_shape=[
            jax.ShapeDtypeStruct((N, K), jnp.float32),
            jax.ShapeDtypeStruct((N, 128), jnp.int32),
            jax.ShapeDtypeStruct((1, 1), jnp.float32),
        ],
        scratch_shapes=[
            pltpu.VMEM((R, 128), jnp.float32),
            pltpu.VMEM((R, 128), jnp.float32),
            pltpu.SMEM((1,), jnp.float32),
        ],
    )(flat, emb)


def _gather_hist(table, idx_flat, hist_K):
    info = plsc.get_sparse_core_info()
    NC, NS, L = info.num_cores, info.num_subcores, info.num_lanes
    NW = NC * NS
    M = idx_flat.shape[0]
    C = table.shape[1]
    per_w = M // NW
    CH = 128
    n_ch = per_w // CH
    mesh = plsc.VectorSubcoreMesh(core_axis_name="c", subcore_axis_name="s")

    @functools.partial(
        pl.kernel,
        out_type=[
            jax.ShapeDtypeStruct((M, C), jnp.float32),
            jax.ShapeDtypeStruct((NW, hist_K), jnp.float32),
        ],
        mesh=mesh,
        scratch_types=[
            pltpu.VMEM((CH,), jnp.int32),
            pltpu.VMEM((CH, C), jnp.float32),
            pltpu.VMEM((hist_K,), jnp.float32),
            pltpu.SemaphoreType.DMA,
        ],
        compiler_params=pltpu.CompilerParams(needs_layout_passes=False),
    )
    def k(table_hbm, idx_hbm, outq_hbm, cnt_hbm, idx_v, rows_v, cnt_v, sem):
        cid = lax.axis_index("c")
        sid = lax.axis_index("s")
        wid = sid * NC + cid
        base = wid * per_w

        def zero_body(i, _):
            cnt_v[pl.ds(i * L, L)] = jnp.zeros((L,), jnp.float32)
            return 0
        lax.fori_loop(0, hist_K // L, zero_body, 0)

        ones = jnp.ones((L,), jnp.float32)
        for c in range(n_ch):
            pltpu.sync_copy(idx_hbm.at[pl.ds(base + c * CH, CH)], idx_v)
            pltpu.async_copy(table_hbm.at[idx_v], rows_v, sem).wait()
            pltpu.sync_copy(rows_v, outq_hbm.at[pl.ds(base + c * CH, CH)])
            for j in range(CH // L):
                v = idx_v[pl.ds(j * L, L)]
                plsc.addupdate_scatter(cnt_v, [v], ones)

        pltpu.sync_copy(cnt_v, cnt_hbm.at[wid])

    return k(table, idx_flat)


def _finalize_body(n_total, cnt_ref, avg_ref, perp_ref):
    counts = jnp.sum(cnt_ref[...], axis=0, keepdims=True)
    avg = counts / n_total
    avg_ref[...] = avg
    ent = jnp.sum(avg * jnp.log(avg + 1e-10))
    perp_ref[...] = jnp.reshape(jnp.exp(-ent), (1, 1))


def _finalize(cnt, n_total):
    NW, K = cnt.shape
    return pl.pallas_call(
        functools.partial(_finalize_body, n_total),
        out_specs=[
            pl.BlockSpec((1, K), lambda: (0, 0)),
            pl.BlockSpec((1, 1), lambda: (0, 0)),
        ],
        out_shape=[
            jax.ShapeDtypeStruct((1, K), jnp.float32),
            jax.ShapeDtypeStruct((1, 1), jnp.float32),
        ],
    )(cnt)


def kernel(inputs, embedding_weight):
    B, T, C = inputs.shape
    K = embedding_weight.shape[0]
    N = B * T
    flat = inputs.reshape(N, C)

    distances, idx_pad, loss = _dist_top3(flat, embedding_weight)
    idx3 = idx_pad[:, :TOPK]
    idx_flat = idx3.reshape(-1)

    quantized_flat, cnt = _gather_hist(embedding_weight, idx_flat, K)
    avg, perp = _finalize(cnt, N)

    loss_out = loss.reshape(())
    quantized_st = quantized_flat.reshape(B, T, TOPK, C)
    perplexity = perp.reshape(())
    avg_probs = avg.reshape(K)
    encoding_indices_out = idx3.reshape(B, T, TOPK)
    distances_out = distances.reshape(B, T, K)
    return (loss_out, quantized_st, perplexity, avg_probs,
            encoding_indices_out, distances_out)


# speedup vs baseline: 1.7842x; 1.0011x over previous
import functools

import jax
import jax.numpy as jnp
from jax import lax
from jax.experimental import pallas as pl
from jax.experimental.pallas import tpu as pltpu
from jax.experimental.pallas import tpu_sc as plsc

TOPK = 3
COMMIT = 0.25
BIG_ID = float(2**30)
CHUNK = 128


def _top3_merge(c, cid):
    mins, sels = [], []
    for t in range(TOPK):
        m = jnp.min(c, axis=1, keepdims=True)
        sel = jnp.min(jnp.where(c == m, cid, BIG_ID), axis=1, keepdims=True)
        mins.append(m)
        sels.append(sel)
        if t < TOPK - 1:
            c = jnp.where(cid == sel, jnp.inf, c)
    return mins, sels


def _pack3(mins, sels, R, base=0):
    lane = lax.broadcasted_iota(jnp.int32, (R, 128), 1)
    nv = jnp.full((R, 128), jnp.inf, jnp.float32)
    ni = jnp.full((R, 128), BIG_ID, jnp.float32)
    for t in range(TOPK - 1, -1, -1):
        nv = jnp.where(lane == base + t, mins[t], nv)
        ni = jnp.where(lane == base + t, sels[t], ni)
    return nv, ni


def _dist_body(nk, x_ref, w_ref, d_ref, idx_ref, loss_ref,
               vals_scr, ids_scr, loss_scr):
    r = pl.program_id(0)
    k = pl.program_id(1)
    R = x_ref.shape[0]
    KB = w_ref.shape[0]
    nch = KB // CHUNK

    x = x_ref[...]
    w = w_ref[...]
    xx = jnp.sum(x * x, axis=1, keepdims=True)
    ww = jnp.sum(w * w, axis=1)[None, :]
    xw2 = lax.dot_general(-2.0 * x, w, (((1,), (1,)), ((), ())),
                          preferred_element_type=jnp.float32)
    d = (xx + ww) + xw2
    d_ref[...] = d

    @pl.when(k == 0)
    def _init():
        vals_scr[...] = jnp.full((R, 128), jnp.inf, jnp.float32)
        ids_scr[...] = jnp.full((R, 128), BIG_ID, jnp.float32)

    @pl.when(jnp.logical_and(r == 0, k == 0))
    def _init_loss():
        loss_scr[0] = 0.0

    gids = (lax.broadcasted_iota(jnp.int32, (R, KB), 1)
            + k * KB).astype(jnp.float32)
    step_vals, step_gids = _top3_merge(d, gids)

    sv, si = _pack3(step_vals, step_gids, R, base=3)
    lane128 = lax.broadcasted_iota(jnp.int32, (R, 128), 1)
    cv = jnp.where(lane128 < TOPK, vals_scr[...], sv)
    ci = jnp.where(lane128 < TOPK, ids_scr[...], si)
    mins, sels = _top3_merge(cv, ci)
    nv, ni = _pack3(mins, sels, R)
    vals_scr[...] = nv
    ids_scr[...] = ni

    @pl.when(k == nk - 1)
    def _emit():
        idx_ref[...] = ids_scr[...].astype(jnp.int32)
        top = jnp.where(lane128 < TOPK, vals_scr[...], 0.0)
        loss_scr[0] += jnp.sum(top)

    @pl.when(jnp.logical_and(r == pl.num_programs(0) - 1, k == nk - 1))
    def _emit_loss():
        n_total = R * pl.num_programs(0)
        v = COMMIT * loss_scr[0] / (n_total * TOPK * x_ref.shape[1])
        loss_ref[...] = jnp.reshape(v, (1, 1))


def _dist_top3(flat, emb, R=2048, KB=1024):
    N, C = flat.shape
    K = emb.shape[0]
    nr, nk = N // R, K // KB
    body = functools.partial(_dist_body, nk)
    return pl.pallas_call(
        body,
        grid=(nr, nk),
        in_specs=[
            pl.BlockSpec((R, C), lambda r, k: (r, 0)),
            pl.BlockSpec((KB, C), lambda r, k: (k, 0)),
        ],
        out_specs=[
            pl.BlockSpec((R, KB), lambda r, k: (r, k)),
            pl.BlockSpec((R, 128), lambda r, k: (r, 0)),
            pl.BlockSpec((1, 1), lambda r, k: (0, 0)),
        ],
        out_shape=[
            jax.ShapeDtypeStruct((N, K), jnp.float32),
            jax.ShapeDtypeStruct((N, 128), jnp.int32),
            jax.ShapeDtypeStruct((1, 1), jnp.float32),
        ],
        scratch_shapes=[
            pltpu.VMEM((R, 128), jnp.float32),
            pltpu.VMEM((R, 128), jnp.float32),
            pltpu.SMEM((1,), jnp.float32),
        ],
    )(flat, emb)


def _gather_hist(table, idx_flat, hist_K):
    info = plsc.get_sparse_core_info()
    NC, NS, L = info.num_cores, info.num_subcores, info.num_lanes
    NW = NC * NS
    M = idx_flat.shape[0]
    C = table.shape[1]
    per_w = M // NW
    CH = 128
    n_ch = per_w // CH
    mesh = plsc.VectorSubcoreMesh(core_axis_name="c", subcore_axis_name="s")

    @functools.partial(
        pl.kernel,
        out_type=[
            jax.ShapeDtypeStruct((M, C), jnp.float32),
            jax.ShapeDtypeStruct((NW, hist_K), jnp.float32),
        ],
        mesh=mesh,
        scratch_types=[
            pltpu.VMEM((per_w,), jnp.int32),
            pltpu.VMEM((CH, C), jnp.float32),
            pltpu.VMEM((CH, C), jnp.float32),
            pltpu.VMEM((hist_K,), jnp.float32),
            pltpu.SemaphoreType.DMA,
            pltpu.SemaphoreType.DMA,
            pltpu.SemaphoreType.DMA,
            pltpu.SemaphoreType.DMA,
        ],
        compiler_params=pltpu.CompilerParams(needs_layout_passes=False),
    )
    def k(table_hbm, idx_hbm, outq_hbm, cnt_hbm,
          idx_all, rows0, rows1, cnt_v, g0, g1, w0, w1):
        cid = lax.axis_index("c")
        sid = lax.axis_index("s")
        wid = sid * NC + cid
        base = wid * per_w
        rows = [rows0, rows1]
        gsem = [g0, g1]
        wsem = [w0, w1]

        pltpu.sync_copy(idx_hbm.at[pl.ds(base, per_w)], idx_all)

        def zero_body(i, _):
            cnt_v[pl.ds(i * L, L)] = jnp.zeros((L,), jnp.float32)
            return 0
        lax.fori_loop(0, hist_K // L, zero_body, 0)

        ones = jnp.ones((L,), jnp.float32)
        gh = [None] * n_ch
        wh = [None] * n_ch
        gh[0] = pltpu.async_copy(
            table_hbm.at[idx_all.at[pl.ds(0, CH)]], rows[0], gsem[0])
        for c in range(n_ch):
            if c + 1 < n_ch:
                if c >= 1:
                    wh[c - 1].wait()
                gh[c + 1] = pltpu.async_copy(
                    table_hbm.at[idx_all.at[pl.ds((c + 1) * CH, CH)]],
                    rows[(c + 1) % 2], gsem[(c + 1) % 2])
            gh[c].wait()
            wh[c] = pltpu.async_copy(
                rows[c % 2], outq_hbm.at[pl.ds(base + c * CH, CH)],
                wsem[c % 2])
            for j in range(CH // L):
                v = idx_all[pl.ds(c * CH + j * L, L)]
                plsc.addupdate_scatter(cnt_v, [v], ones)

        wh[n_ch - 2].wait()
        wh[n_ch - 1].wait()
        pltpu.sync_copy(cnt_v, cnt_hbm.at[wid])

    return k(table, idx_flat)


def _finalize_body(n_total, cnt_ref, avg_ref, perp_ref):
    counts = jnp.sum(cnt_ref[...], axis=0, keepdims=True)
    avg = counts / n_total
    avg_ref[...] = avg
    ent = jnp.sum(avg * jnp.log(avg + 1e-10))
    perp_ref[...] = jnp.reshape(jnp.exp(-ent), (1, 1))


def _finalize(cnt, n_total):
    NW, K = cnt.shape
    return pl.pallas_call(
        functools.partial(_finalize_body, n_total),
        out_specs=[
            pl.BlockSpec((1, K), lambda: (0, 0)),
            pl.BlockSpec((1, 1), lambda: (0, 0)),
        ],
        out_shape=[
            jax.ShapeDtypeStruct((1, K), jnp.float32),
            jax.ShapeDtypeStruct((1, 1), jnp.float32),
        ],
    )(cnt)


def kernel(inputs, embedding_weight):
    B, T, C = inputs.shape
    K = embedding_weight.shape[0]
    N = B * T
    flat = inputs.reshape(N, C)

    distances, idx_pad, loss = _dist_top3(flat, embedding_weight)
    idx3 = idx_pad[:, :TOPK]
    idx_flat = idx3.reshape(-1)

    quantized_flat, cnt = _gather_hist(embedding_weight, idx_flat, K)
    avg, perp = _finalize(cnt, N)

    loss_out = loss.reshape(())
    quantized_st = quantized_flat.reshape(B, T, TOPK, C)
    perplexity = perp.reshape(())
    avg_probs = avg.reshape(K)
    encoding_indices_out = idx3.reshape(B, T, TOPK)
    distances_out = distances.reshape(B, T, K)
    return (loss_out, quantized_st, perplexity, avg_probs,
            encoding_indices_out, distances_out)
